# baseline (speedup 1.0000x reference)
import functools

import jax
import jax.numpy as jnp
import numpy as np
from jax import lax
from jax.experimental import pallas as pl
from jax.experimental.pallas import tpu as pltpu
from jax.experimental.pallas import tpu_sc as plsc

DIM = 32
NUM_CORES = 2
NUM_SUBCORES = 16
NW = NUM_CORES * NUM_SUBCORES
EPS = 1e-4
INV_SQRT_DIM = float(1.0 / np.sqrt(DIM))

_TBLK = 8192
_TSUB = 128


def _tnorm_body(wt_ref, out_ref):
    t = wt_ref[...]
    s = jnp.sum(t * t, axis=0)
    scale = 1.0 / (EPS + jnp.sqrt(s) * INV_SQRT_DIM)
    t = t * scale[None, :]
    na = _TSUB // DIM
    e = lax.broadcasted_iota(jnp.int32, (_TSUB, _TSUB), 0)
    c = lax.broadcasted_iota(jnp.int32, (_TSUB, _TSUB), 1)
    selbig = jnp.where(e == na * (c % DIM) + c // DIM, jnp.float32(1),
                       jnp.float32(0))
    for j in range(_TBLK // _TSUB):
        tj = lax.slice(t, (0, j * _TSUB), (DIM, (j + 1) * _TSUB))
        m = lax.dot_general(tj, selbig, (((1,), (0,)), ((), ())),
                            preferred_element_type=jnp.float32)
        parts = []
        for a in range(na):
            ma = lax.slice(m, (0, a * DIM), (DIM, (a + 1) * DIM))
            parts.append(jnp.transpose(ma))
        u = jnp.concatenate(parts, axis=1)
        out_ref[pl.ds(j * _TSUB * DIM, _TSUB * DIM)] = jnp.reshape(u, (_TSUB * DIM,))


@functools.lru_cache(maxsize=None)
def _build_tnorm(nemb):
    nblk = (nemb + _TBLK - 1) // _TBLK
    return pl.pallas_call(
        _tnorm_body,
        grid=(nblk,),
        in_specs=[pl.BlockSpec((DIM, _TBLK), lambda i: (0, i))],
        out_specs=pl.BlockSpec((_TBLK * DIM,), lambda i: (i,)),
        out_shape=jax.ShapeDtypeStruct((nemb * DIM,), jnp.float32),
    )


@functools.lru_cache(maxsize=None)
def _build(nb, nt):
    nj = nb // 128
    pairs = nt * nj
    ppw = pairs // NW
    assert pairs % NW == 0 and nj == 128
    mesh = plsc.VectorSubcoreMesh(core_axis_name="c", subcore_axis_name="s")
    srow = DIM + 1

    @functools.partial(
        pl.kernel,
        out_type=jax.ShapeDtypeStruct((nt, DIM // 8, nj, 8, 128), jnp.float32),
        mesh=mesh,
        scratch_types=[
            pltpu.VMEM((128,), jnp.int32),
            pltpu.VMEM((128, DIM), jnp.float32),
            pltpu.VMEM((128, srow), jnp.float32),
            pltpu.VMEM((DIM, 128), jnp.float32),
            pltpu.SemaphoreType.DMA,
        ],
        compiler_params=pltpu.CompilerParams(
            needs_layout_passes=False, use_tc_tiling_on_sc=False),
    )
    def impl(idx_hbm, table_hbm, out_hbm, idx_v, rows32_v, rows_v, zt_v, sem):
        wid = lax.axis_index("s") * NUM_CORES + lax.axis_index("c")
        p_base = wid * ppw
        riota = jnp.arange(16, dtype=jnp.int32)

        def pair_body(ci, carry):
            p = p_base + ci
            t = lax.shift_right_logical(p, 7)
            j = lax.bitwise_and(p, 127)
            off = t * nb + j * 128
            pltpu.sync_copy(idx_hbm.at[pl.ds(off, 128)], idx_v)
            pltpu.async_copy(table_hbm.at[idx_v], rows32_v, sem).wait()
            for r in range(128):
                rows_v[r, pl.ds(0, 16)] = rows32_v[r, pl.ds(0, 16)]
                rows_v[r, pl.ds(16, 16)] = rows32_v[r, pl.ds(16, 16)]
            for d in range(DIM):
                colid = jnp.full((16,), d, dtype=jnp.int32)
                for g in range(8):
                    rowid = riota + (g * 16)
                    vec = plsc.load_gather(rows_v, [rowid, colid])
                    zt_v[d, pl.ds(g * 16, 16)] = vec
            for k in range(DIM // 8):
                pltpu.sync_copy(zt_v.at[pl.ds(k * 8, 8)], out_hbm.at[t, k, j])
            return carry

        lax.fori_loop(0, ppw, pair_body, 0)

    return impl


def kernel(x, weight):
    nb, nt = x.shape
    nemb = weight.shape[0]
    xtf = jnp.reshape(jnp.transpose(x), (nb * nt,)).astype(jnp.int32)
    wn = jnp.reshape(_build_tnorm(nemb)(weight.T), (nemb, DIM))
    z = _build(nb, nt)(xtf, wn)
    return jnp.reshape(jnp.transpose(z, (2, 4, 0, 1, 3)), (nb, nt, DIM))


# speedup vs baseline: 2.0888x; 2.0888x over previous
import functools

import jax
import jax.numpy as jnp
import numpy as np
from jax import lax
from jax.experimental import pallas as pl
from jax.experimental.pallas import tpu as pltpu
from jax.experimental.pallas import tpu_sc as plsc

DIM = 32
NUM_CORES = 2
NUM_SUBCORES = 16
NW = NUM_CORES * NUM_SUBCORES
EPS = 1e-4
INV_SQRT_DIM = float(1.0 / np.sqrt(DIM))

_TBLK = 8192
_TSUB = 128


def _tnorm_body(wt_ref, out_ref):
    t = wt_ref[...]
    s = jnp.sum(t * t, axis=0)
    scale = 1.0 / (EPS + jnp.sqrt(s) * INV_SQRT_DIM)
    t = t * scale[None, :]
    e = lax.broadcasted_iota(jnp.int32, (_TSUB, DIM), 0)
    r = lax.broadcasted_iota(jnp.int32, (_TSUB, DIM), 1)
    sels = [jnp.where(e == (_TSUB // DIM) * r + a, jnp.float32(1), jnp.float32(0))
            for a in range(_TSUB // DIM)]
    for j in range(_TBLK // _TSUB):
        tj = lax.slice(t, (0, j * _TSUB), (DIM, (j + 1) * _TSUB))
        parts = []
        for ea in sels:
            ma = lax.dot_general(tj, ea, (((1,), (0,)), ((), ())),
                                 preferred_element_type=jnp.float32)
            parts.append(jnp.transpose(ma))
        u = jnp.concatenate(parts, axis=1)
        out_ref[pl.ds(j * _TSUB * DIM, _TSUB * DIM)] = jnp.reshape(u, (_TSUB * DIM,))


@functools.lru_cache(maxsize=None)
def _build_tnorm(nemb):
    nblk = (nemb + _TBLK - 1) // _TBLK
    return pl.pallas_call(
        _tnorm_body,
        grid=(nblk,),
        in_specs=[pl.BlockSpec((DIM, _TBLK), lambda i: (0, i))],
        out_specs=pl.BlockSpec((_TBLK * DIM,), lambda i: (i,)),
        out_shape=jax.ShapeDtypeStruct((nemb * DIM,), jnp.float32),
    )


@functools.lru_cache(maxsize=None)
def _build(nb, nt):
    nj = nb // 128
    pairs = nt * nj
    ppw = pairs // NW
    assert pairs % NW == 0 and nj == 128
    mesh = plsc.VectorSubcoreMesh(core_axis_name="c", subcore_axis_name="s")
    srow = DIM + 1

    @functools.partial(
        pl.kernel,
        out_type=jax.ShapeDtypeStruct((nt, DIM // 8, nj, 8, 128), jnp.float32),
        mesh=mesh,
        scratch_types=[
            pltpu.VMEM((128,), jnp.int32),
            pltpu.VMEM((128, DIM), jnp.float32),
            pltpu.VMEM((128, srow), jnp.float32),
            pltpu.VMEM((DIM, 128), jnp.float32),
            pltpu.SemaphoreType.DMA,
        ],
        compiler_params=pltpu.CompilerParams(
            needs_layout_passes=False, use_tc_tiling_on_sc=False),
    )
    def impl(idx_hbm, table_hbm, out_hbm, idx_v, rows32_v, rows_v, zt_v, sem):
        wid = lax.axis_index("s") * NUM_CORES + lax.axis_index("c")
        p_base = wid * ppw
        riota = jnp.arange(16, dtype=jnp.int32)

        def pair_body(ci, carry):
            p = p_base + ci
            t = lax.shift_right_logical(p, 7)
            j = lax.bitwise_and(p, 127)
            off = t * nb + j * 128
            pltpu.sync_copy(idx_hbm.at[pl.ds(off, 128)], idx_v)
            pltpu.async_copy(table_hbm.at[idx_v], rows32_v, sem).wait()
            for r in range(128):
                rows_v[r, pl.ds(0, 16)] = rows32_v[r, pl.ds(0, 16)]
                rows_v[r, pl.ds(16, 16)] = rows32_v[r, pl.ds(16, 16)]
            for d in range(DIM):
                colid = jnp.full((16,), d, dtype=jnp.int32)
                for g in range(8):
                    rowid = riota + (g * 16)
                    vec = plsc.load_gather(rows_v, [rowid, colid])
                    zt_v[d, pl.ds(g * 16, 16)] = vec
            for k in range(DIM // 8):
                pltpu.sync_copy(zt_v.at[pl.ds(k * 8, 8)], out_hbm.at[t, k, j])
            return carry

        lax.fori_loop(0, ppw, pair_body, 0)

    return impl


def kernel(x, weight):
    nb, nt = x.shape
    nemb = weight.shape[0]
    xtf = jnp.reshape(jnp.transpose(x), (nb * nt,)).astype(jnp.int32)
    wn = jnp.reshape(_build_tnorm(nemb)(weight.T), (nemb, DIM))
    z = _build(nb, nt)(xtf, wn)
    return jnp.reshape(jnp.transpose(z, (2, 4, 0, 1, 3)), (nb, nt, DIM))
